# baseline (speedup 1.0000x reference)
import functools

import jax
import jax.numpy as jnp
from jax import lax
from jax.experimental import pallas as pl
from jax.experimental.pallas import tpu as pltpu
from jax.experimental.pallas import tpu_sc as plsc

B = 16384
D = 128
NC = 2
NS = 16
L = 16
NW = NC * NS
BPW = B // NW
CH = 128
NCH = BPW // CH
G = 16
NG = CH // G


def _sc_diff_kernel(user_hbm, pos_hbm, neg_hbm, utab_hbm, itab_hbm, out_hbm,
                    uidx_v, pidx_v, nidx_v, u_v, p_v, n_v, scr_v, dot_v, sem):
    c = lax.axis_index("c")
    s = lax.axis_index("s")
    wid = s * NC + c

    pltpu.sync_copy(user_hbm.at[wid], uidx_v)
    pltpu.sync_copy(pos_hbm.at[wid], pidx_v)
    pltpu.sync_copy(neg_hbm.at[wid], nidx_v)

    lanes = lax.iota(jnp.int32, L)

    for ch in range(NCH):
        cu = pltpu.async_copy(utab_hbm.at[uidx_v.at[ch]], u_v, sem)
        cp = pltpu.async_copy(itab_hbm.at[pidx_v.at[ch]], p_v, sem)
        cn = pltpu.async_copy(itab_hbm.at[nidx_v.at[ch]], n_v, sem)
        cu.wait()
        cp.wait()
        cn.wait()

        def group_body(g, carry, ch=ch):
            dot = jnp.zeros((L,), jnp.float32)
            for r in range(G):
                acc = None
                for j in range(D // L):
                    uu = u_v[g * G + r, pl.ds(j * L, L)]
                    pp = p_v[g * G + r, pl.ds(j * L, L)]
                    nn = n_v[g * G + r, pl.ds(j * L, L)]
                    t = uu * (pp - nn)
                    acc = t if acc is None else acc + t
                dot = jnp.where(lanes == r, jnp.sum(acc), dot)
            dot_v[pl.ds(ch * CH + g * G, G)] = dot
            return carry

        lax.fori_loop(0, NG, group_body, 0)

    pltpu.sync_copy(dot_v, out_hbm.at[wid])


def _diff_on_sc(user, pos_item, neg_item, user_table, item_table):
    mesh = plsc.VectorSubcoreMesh(core_axis_name="c", subcore_axis_name="s")
    kfn = pl.kernel(
        _sc_diff_kernel,
        mesh=mesh,
        compiler_params=pltpu.CompilerParams(needs_layout_passes=False),
        out_type=jax.ShapeDtypeStruct((NW, BPW), jnp.float32),
        scratch_types=[
            pltpu.VMEM((NCH, CH), jnp.int32),
            pltpu.VMEM((NCH, CH), jnp.int32),
            pltpu.VMEM((NCH, CH), jnp.int32),
            pltpu.VMEM((CH, D), jnp.float32),
            pltpu.VMEM((CH, D), jnp.float32),
            pltpu.VMEM((CH, D), jnp.float32),
            pltpu.VMEM((G * L,), jnp.float32),
            pltpu.VMEM((BPW,), jnp.float32),
            pltpu.SemaphoreType.DMA,
        ],
    )
    diff = kfn(
        user.reshape(NW, NCH, CH),
        pos_item.reshape(NW, NCH, CH),
        neg_item.reshape(NW, NCH, CH),
        user_table,
        item_table,
    )
    return diff.reshape(B)


def _loss_body(x_ref, o_ref):
    x = x_ref[...]
    t = -jnp.log(1e-8 + jax.nn.sigmoid(x))
    o_ref[0, 0] = jnp.sum(t) * (1.0 / B)


def _loss_on_tc(diff):
    out = pl.pallas_call(
        _loss_body,
        out_shape=jax.ShapeDtypeStruct((1, 1), jnp.float32),
        out_specs=pl.BlockSpec(memory_space=pltpu.SMEM),
    )(diff.reshape(B // D, D))
    return out[0, 0]


@jax.jit
def kernel(user, pos_item, neg_item, user_table, item_table):
    diff = _diff_on_sc(user, pos_item, neg_item, user_table, item_table)
    return _loss_on_tc(diff)
